# baseline (speedup 1.0000x reference)
import functools

import jax
import jax.numpy as jnp
from jax import lax
from jax.experimental import pallas as pl
from jax.experimental.pallas import tpu as pltpu
from jax.experimental.pallas import tpu_sc as plsc

BATCH = 4096
FIELD_DIM = 26
VOCAB = 100000
EMBED = 64
NNZ = BATCH * FIELD_DIM

NCORE = 2
NSUB = 16
NW = NCORE * NSUB
CPW = 4
NCHUNK = NW * CPW
CHUNK_B = BATCH // NCHUNK
CROWS = CHUNK_B * FIELD_DIM
K = 128
ZROWS = 104
NPAD = NNZ + 17 * K
NB = NPAD // K


def _mesh_kernel():
    mesh = plsc.VectorSubcoreMesh(core_axis_name="c", subcore_axis_name="s")

    @functools.partial(
        pl.kernel,
        mesh=mesh,
        out_type=jax.ShapeDtypeStruct((NNZ * EMBED,), jnp.float32),
        compiler_params=pltpu.CompilerParams(use_tc_tiling_on_sc=False),
        scratch_types=[
            pltpu.VMEM((16,), jnp.int32),
            pltpu.VMEM((3, K), jnp.int32),
            pltpu.VMEM((3, K), jnp.int32),
            pltpu.VMEM((K,), jnp.float32),
            pltpu.VMEM((K,), jnp.float32),
            pltpu.VMEM((K,), jnp.int32),
            pltpu.VMEM((K, EMBED), jnp.float32),
            pltpu.VMEM((K, EMBED), jnp.float32),
            pltpu.VMEM(((CROWS + 8) * EMBED,), jnp.float32),
            pltpu.SemaphoreType.DMA,
            pltpu.SemaphoreType.DMA,
            pltpu.SemaphoreType.DMA,
            pltpu.SemaphoreType.DMA,
            pltpu.SemaphoreType.DMA,
            pltpu.SemaphoreType.DMA,
        ],
    )
    def k(meta_hbm, pk_hbm, val_hbm, table_hbm, out_hbm,
          meta_v, pk_v0, pk_v1, val_v0, val_v1, lrow_v, rows_v0, rows_v1,
          acc_v, sem_pk0, sem_pk1, sem_val0, sem_val1,
          sem_g0, sem_g1):
        cid = lax.axis_index("c")
        sid = lax.axis_index("s")
        wid = sid * NCORE + cid
        z16 = jnp.zeros((16,), jnp.float32)
        lane16 = lax.broadcasted_iota(jnp.int32, (16,), 0)
        pk_v = (pk_v0, pk_v1)
        val_v = (val_v0, val_v1)
        rows_v = (rows_v0, rows_v1)
        sem_pk = (sem_pk0, sem_pk1)
        sem_val = (sem_val0, sem_val1)
        sem_g = (sem_g0, sem_g1)

        pltpu.sync_copy(meta_hbm.at[wid], meta_v)
        mv = meta_v[...]

        for ck in range(CPW):
            g = wid * CPW + ck
            r0 = g * CROWS
            s = mv[ck]
            e = mv[ck + 1]
            b0 = s // K
            n = (e - b0 * K + K - 1) // K

            def zrow(i, _):
                zc = i * 64
                for u in range(4):
                    acc_v[pl.ds(zc + u * 16, 16)] = z16
                return 0

            lax.fori_loop(0, CROWS, zrow, 0)

            def blk(i):
                return b0 + i

            def start_in(i, sl):
                pltpu.async_copy(pk_hbm.at[blk(i)], pk_v[sl], sem_pk[sl])
                pltpu.async_copy(val_hbm.at[blk(i)], val_v[sl], sem_val[sl])

            def wait_pk(sl):
                pltpu.make_async_copy(
                    pk_hbm.at[0], pk_v[sl], sem_pk[sl]).wait()

            def wait_val(sl):
                pltpu.make_async_copy(
                    val_hbm.at[0], val_v[sl], sem_val[sl]).wait()

            def start_gather(sl):
                pltpu.async_copy(
                    table_hbm.at[pk_v[sl].at[2]], rows_v[sl], sem_g[sl])

            def wait_gather(sl):
                pltpu.make_async_copy(
                    table_hbm.at[pl.ds(0, K)], rows_v[sl], sem_g[sl]).wait()

            @pl.when(n > 0)
            def _():
                start_in(0, 0)
                wait_pk(0)
                start_gather(0)

            @pl.when(n > 1)
            def _():
                start_in(1, 1)

            def body(i, sl):
                so = 1 - sl
                for j in range(K // 16):
                    s16 = pl.ds(j * 16, 16)
                    lr = (pk_v[sl][0, s16] * FIELD_DIM
                          + pk_v[sl][1, s16] - r0)
                    inr = (lr >= 0) & (lr < CROWS)
                    lrow_v[s16] = jnp.where(inr, lr, CROWS) * EMBED
                wait_gather(sl)

                @pl.when(i + 1 < n)
                def _():
                    wait_pk(so)
                    start_gather(so)

                wait_val(sl)

                def wrow(j, _):
                    s16 = pl.ds(j * 16, 16)
                    v16 = val_v[sl][s16]
                    l16 = lrow_v[s16]
                    for i2 in range(16):
                        sv = v16[i2]
                        base = l16[i2]
                        r = j * 16 + i2
                        for c in range(EMBED // 16):
                            dst = pl.ds(base + c * 16, 16)
                            w = rows_v[sl][r, pl.ds(c * 16, 16)] * sv
                            acc_v[dst] = acc_v[dst] + w
                    return 0

                lax.fori_loop(0, K // 16, wrow, 0)

                @pl.when(i + 2 < n)
                def _():
                    start_in(i + 2, sl)

            def pair(t, _):
                for sl in range(2):
                    i = 2 * t + sl

                    @pl.when(i < n)
                    def _():
                        body(i, sl)

                return 0

            lax.fori_loop(0, (n + 1) // 2, pair, 0)

            pltpu.sync_copy(
                acc_v.at[pl.ds(0, CROWS * EMBED)],
                out_hbm.at[pl.ds(r0 * EMBED, CROWS * EMBED)])

    return k


_sc_call = _mesh_kernel()


@jax.jit
def _run(meta, packed, vals, table):
    return _sc_call(meta, packed, vals, table)


def kernel(feature_embedding, field_idx, field_sub_idx, feature_idx,
           feature_vals, batch_idx):
    del field_sub_idx
    i32 = jnp.int32
    pad = NPAD - NNZ
    bi = jnp.concatenate([batch_idx.astype(i32), jnp.full((pad,), BATCH, i32)])
    fi = jnp.concatenate([field_idx.astype(i32), jnp.zeros((pad,), i32)])
    fx = jnp.concatenate([feature_idx.astype(i32), jnp.zeros((pad,), i32)])
    fv = jnp.concatenate(
        [feature_vals, jnp.zeros((pad,), jnp.float32)]).reshape(NB, K)
    packed = jnp.stack([bi, fi, fx]).reshape(3, NB, K).transpose(1, 0, 2)
    bounds = jnp.searchsorted(
        batch_idx,
        jnp.arange(0, BATCH + 1, CHUNK_B, dtype=i32)).astype(i32)
    cols = CPW * jnp.arange(NW, dtype=i32)[:, None] + jnp.arange(
        16, dtype=i32)[None, :]
    meta = bounds[jnp.minimum(cols, NCHUNK)]
    out = _run(meta, packed, fv, feature_embedding)
    return out.reshape(NNZ, EMBED)


# speedup vs baseline: 1.4126x; 1.4126x over previous
import functools

import jax
import jax.numpy as jnp
from jax import lax
from jax.experimental import pallas as pl
from jax.experimental.pallas import tpu as pltpu
from jax.experimental.pallas import tpu_sc as plsc

BATCH = 4096
FIELD_DIM = 26
VOCAB = 100000
EMBED = 64
NNZ = BATCH * FIELD_DIM

NCORE = 2
NSUB = 16
CPC = 2
NCHUNK = NCORE * CPC
CHUNK_B = BATCH // NCHUNK
CROWS = CHUNK_B * FIELD_DIM
TROWS = CROWS // NSUB
K = 128
ZROWS = 104
NPAD = NNZ + 17 * K
NB = NPAD // K


def _mesh_kernel():
    mesh = plsc.VectorSubcoreMesh(core_axis_name="c", subcore_axis_name="s")

    @functools.partial(
        pl.kernel,
        mesh=mesh,
        out_type=jax.ShapeDtypeStruct((NNZ, EMBED), jnp.float32),
        compiler_params=pltpu.CompilerParams(use_tc_tiling_on_sc=False),
        scratch_types=[
            pltpu.VMEM((32,), jnp.int32),
            pltpu.VMEM((3, K), jnp.int32),
            pltpu.VMEM((3, K), jnp.int32),
            pltpu.VMEM((K,), jnp.float32),
            pltpu.VMEM((K,), jnp.float32),
            pltpu.VMEM((K,), jnp.int32),
            pltpu.VMEM((K, EMBED), jnp.float32),
            pltpu.VMEM((K, EMBED), jnp.float32),
            pltpu.VMEM((ZROWS, EMBED), jnp.float32),
            pltpu.VMEM_SHARED((CROWS + 8, EMBED), jnp.float32),
            pltpu.SemaphoreType.DMA,
            pltpu.SemaphoreType.DMA,
            pltpu.SemaphoreType.DMA,
            pltpu.SemaphoreType.DMA,
            pltpu.SemaphoreType.DMA,
            pltpu.SemaphoreType.DMA,
        ],
    )
    def k(meta_hbm, pk_hbm, val_hbm, table_hbm, out_hbm,
          meta_v, pk_v0, pk_v1, val_v0, val_v1, lrow_v, rows_v0, rows_v1,
          zbuf, acc_sh, sem_pk0, sem_pk1, sem_val0, sem_val1,
          sem_g0, sem_g1):
        cid = lax.axis_index("c")
        sid = lax.axis_index("s")
        wid = sid * NCORE + cid
        z16 = jnp.zeros((16,), jnp.float32)
        pk_v = (pk_v0, pk_v1)
        val_v = (val_v0, val_v1)
        rows_v = (rows_v0, rows_v1)
        sem_pk = (sem_pk0, sem_pk1)
        sem_val = (sem_val0, sem_val1)
        sem_g = (sem_g0, sem_g1)

        def zrow(i, _):
            for c in range(EMBED // 16):
                zbuf[i, pl.ds(c * 16, 16)] = z16
            return 0

        lax.fori_loop(0, ZROWS, zrow, 0)

        pltpu.sync_copy(meta_hbm.at[wid], meta_v)
        myrow0 = sid * TROWS

        def chunk(ck, _):
            vv = meta_v[pl.ds(ck * 8, 16)]
            s = vv[0]
            e = vv[8]
            r0 = (cid * CPC + ck) * CROWS
            b0 = s // K
            nt = (e - b0 * K + K - 1) // K
            n = jnp.maximum(0, (nt - sid + NSUB - 1) // NSUB)

            for zb in range(TROWS // ZROWS):
                pltpu.sync_copy(
                    zbuf, acc_sh.at[pl.ds(myrow0 + zb * ZROWS, ZROWS)])
            plsc.subcore_barrier()

            def blk(i):
                return b0 + sid + i * NSUB

            def start_in(i, sl):
                pltpu.async_copy(pk_hbm.at[blk(i)], pk_v[sl], sem_pk[sl])
                pltpu.async_copy(val_hbm.at[blk(i)], val_v[sl], sem_val[sl])

            def wait_pk(sl):
                pltpu.make_async_copy(
                    pk_hbm.at[0], pk_v[sl], sem_pk[sl]).wait()

            def wait_val(sl):
                pltpu.make_async_copy(
                    val_hbm.at[0], val_v[sl], sem_val[sl]).wait()

            def start_gather(sl):
                pltpu.async_copy(
                    table_hbm.at[pk_v[sl].at[2]], rows_v[sl], sem_g[sl])

            def wait_gather(sl):
                pltpu.make_async_copy(
                    table_hbm.at[pl.ds(0, K)], rows_v[sl], sem_g[sl]).wait()

            @pl.when(n > 0)
            def _():
                start_in(0, 0)
                wait_pk(0)
                start_gather(0)

            @pl.when(n > 1)
            def _():
                start_in(1, 1)

            def body(i, sl):
                so = 1 - sl
                for j in range(K // 16):
                    s16 = pl.ds(j * 16, 16)
                    lr = (pk_v[sl][0, s16] * FIELD_DIM
                          + pk_v[sl][1, s16] - r0)
                    inr = (lr >= 0) & (lr < CROWS)
                    lrow_v[s16] = jnp.where(inr, lr, CROWS)
                wait_gather(sl)

                @pl.when(i + 1 < n)
                def _():
                    wait_pk(so)
                    start_gather(so)

                wait_val(sl)

                for j in range(K // 16):
                    v16 = val_v[sl][pl.ds(j * 16, 16)]
                    for i2 in range(16):
                        sv = v16[i2]
                        r = j * 16 + i2
                        for c in range(EMBED // 16):
                            sc = pl.ds(c * 16, 16)
                            rows_v[sl][r, sc] = rows_v[sl][r, sc] * sv

                pltpu.sync_copy(rows_v[sl], acc_sh.at[lrow_v], add=True)

                @pl.when(i + 2 < n)
                def _():
                    start_in(i + 2, sl)

            def pair(t, _):
                for sl in range(2):
                    i = 2 * t + sl

                    @pl.when(i < n)
                    def _():
                        body(i, sl)

                return 0

            lax.fori_loop(0, (n + 1) // 2, pair, 0)
            plsc.subcore_barrier()

            pltpu.sync_copy(
                acc_sh.at[pl.ds(myrow0, TROWS)],
                out_hbm.at[pl.ds(r0 + myrow0, TROWS)])
            return 0

        lax.fori_loop(0, CPC, chunk, 0)

    return k


_sc_call = _mesh_kernel()


@jax.jit
def _run(meta, packed, vals, table):
    return _sc_call(meta, packed, vals, table)


def kernel(feature_embedding, field_idx, field_sub_idx, feature_idx,
           feature_vals, batch_idx):
    del field_sub_idx
    i32 = jnp.int32
    pad = NPAD - NNZ
    bi = jnp.concatenate([batch_idx.astype(i32), jnp.full((pad,), BATCH, i32)])
    fi = jnp.concatenate([field_idx.astype(i32), jnp.zeros((pad,), i32)])
    fx = jnp.concatenate([feature_idx.astype(i32), jnp.zeros((pad,), i32)])
    fv = jnp.concatenate(
        [feature_vals, jnp.zeros((pad,), jnp.float32)]).reshape(NB, K)
    packed = jnp.stack([bi, fi, fx]).reshape(3, NB, K).transpose(1, 0, 2)
    bounds = jnp.searchsorted(
        batch_idx,
        jnp.arange(0, BATCH + 1, CHUNK_B, dtype=i32)).astype(i32)
    c_of_w = jnp.arange(32, dtype=i32) % NCORE
    cols = CPC * c_of_w[:, None] + (jnp.arange(32, dtype=i32) // 8)[None, :]
    meta = bounds[jnp.minimum(cols, NCHUNK)]
    return _run(meta, packed, fv, feature_embedding)


# speedup vs baseline: 1.4439x; 1.0222x over previous
import functools

import jax
import jax.numpy as jnp
from jax import lax
from jax.experimental import pallas as pl
from jax.experimental.pallas import tpu as pltpu
from jax.experimental.pallas import tpu_sc as plsc

BATCH = 4096
FIELD_DIM = 26
VOCAB = 100000
EMBED = 64
NNZ = BATCH * FIELD_DIM

NCORE = 2
NSUB = 16
CPC = 2
NCHUNK = NCORE * CPC
CHUNK_B = BATCH // NCHUNK
CROWS = CHUNK_B * FIELD_DIM
TROWS = CROWS // NSUB
K = 128
ZROWS = 104
NPAD = NNZ + 17 * K
NB = NPAD // K


def _mesh_kernel():
    mesh = plsc.VectorSubcoreMesh(core_axis_name="c", subcore_axis_name="s")

    @functools.partial(
        pl.kernel,
        mesh=mesh,
        out_type=jax.ShapeDtypeStruct((NNZ, EMBED), jnp.float32),
        compiler_params=pltpu.CompilerParams(use_tc_tiling_on_sc=False),
        scratch_types=[
            pltpu.VMEM((32,), jnp.int32),
            pltpu.VMEM((3, K), jnp.int32),
            pltpu.VMEM((3, K), jnp.int32),
            pltpu.VMEM((K,), jnp.float32),
            pltpu.VMEM((K,), jnp.float32),
            pltpu.VMEM((K,), jnp.int32),
            pltpu.VMEM((K,), jnp.int32),
            pltpu.VMEM((K, EMBED), jnp.float32),
            pltpu.VMEM((K, EMBED), jnp.float32),
            pltpu.VMEM((ZROWS, EMBED), jnp.float32),
            pltpu.VMEM_SHARED((CROWS + 8, EMBED), jnp.float32),
            pltpu.SemaphoreType.DMA,
            pltpu.SemaphoreType.DMA,
            pltpu.SemaphoreType.DMA,
            pltpu.SemaphoreType.DMA,
            pltpu.SemaphoreType.DMA,
            pltpu.SemaphoreType.DMA,
            pltpu.SemaphoreType.DMA,
            pltpu.SemaphoreType.DMA,
        ],
    )
    def k(meta_hbm, pk_hbm, val_hbm, table_hbm, out_hbm,
          meta_v, pk_v0, pk_v1, val_v0, val_v1, lrow_v0, lrow_v1,
          rows_v0, rows_v1, zbuf, acc_sh, sem_pk0, sem_pk1,
          sem_val0, sem_val1, sem_g0, sem_g1, sem_sc0, sem_sc1):
        cid = lax.axis_index("c")
        sid = lax.axis_index("s")
        wid = sid * NCORE + cid
        z16 = jnp.zeros((16,), jnp.float32)
        pk_v = (pk_v0, pk_v1)
        val_v = (val_v0, val_v1)
        rows_v = (rows_v0, rows_v1)
        sem_pk = (sem_pk0, sem_pk1)
        sem_val = (sem_val0, sem_val1)
        sem_g = (sem_g0, sem_g1)
        sem_sc = (sem_sc0, sem_sc1)
        lrow_v = (lrow_v0, lrow_v1)

        def zrow(i, _):
            for c in range(EMBED // 16):
                zbuf[i, pl.ds(c * 16, 16)] = z16
            return 0

        lax.fori_loop(0, ZROWS, zrow, 0)

        pltpu.sync_copy(meta_hbm.at[wid], meta_v)
        myrow0 = sid * TROWS

        def chunk(ck, _):
            vv = meta_v[pl.ds(ck * 8, 16)]
            s = vv[0]
            e = vv[8]
            r0 = (cid * CPC + ck) * CROWS
            b0 = s // K
            nt = (e - b0 * K + K - 1) // K
            n = jnp.maximum(0, (nt - sid + NSUB - 1) // NSUB)

            for zb in range(TROWS // ZROWS):
                pltpu.sync_copy(
                    zbuf, acc_sh.at[pl.ds(myrow0 + zb * ZROWS, ZROWS)])
            plsc.subcore_barrier()

            def blk(i):
                return b0 + sid + i * NSUB

            def start_in(i, sl):
                pltpu.async_copy(pk_hbm.at[blk(i)], pk_v[sl], sem_pk[sl])
                pltpu.async_copy(val_hbm.at[blk(i)], val_v[sl], sem_val[sl])

            def wait_pk(sl):
                pltpu.make_async_copy(
                    pk_hbm.at[0], pk_v[sl], sem_pk[sl]).wait()

            def wait_val(sl):
                pltpu.make_async_copy(
                    val_hbm.at[0], val_v[sl], sem_val[sl]).wait()

            def start_gather(sl):
                pltpu.async_copy(
                    table_hbm.at[pk_v[sl].at[2]], rows_v[sl], sem_g[sl])

            def wait_gather(sl):
                pltpu.make_async_copy(
                    table_hbm.at[pl.ds(0, K)], rows_v[sl], sem_g[sl]).wait()

            def wait_sc(sl):
                pltpu.make_async_copy(
                    table_hbm.at[pl.ds(0, K)], rows_v[sl], sem_sc[sl]).wait()

            @pl.when(n > 0)
            def _():
                start_in(0, 0)
                wait_pk(0)
                start_gather(0)

            @pl.when(n > 1)
            def _():
                start_in(1, 1)

            def body(i, sl):
                so = 1 - sl
                for j in range(K // 16):
                    s16 = pl.ds(j * 16, 16)
                    lr = (pk_v[sl][0, s16] * FIELD_DIM
                          + pk_v[sl][1, s16] - r0)
                    inr = (lr >= 0) & (lr < CROWS)
                    lrow_v[sl][s16] = jnp.where(inr, lr, CROWS)
                wait_gather(sl)

                @pl.when(i + 1 < n)
                def _():
                    @pl.when(i >= 1)
                    def _():
                        wait_sc(so)

                    wait_pk(so)
                    start_gather(so)

                wait_val(sl)

                for j in range(K // 16):
                    v16 = val_v[sl][pl.ds(j * 16, 16)]
                    for i2 in range(16):
                        sv = v16[i2]
                        r = j * 16 + i2
                        for c in range(EMBED // 16):
                            sc = pl.ds(c * 16, 16)
                            rows_v[sl][r, sc] = rows_v[sl][r, sc] * sv

                pltpu.async_copy(
                    rows_v[sl], acc_sh.at[lrow_v[sl]], sem_sc[sl],
                    add=True)

                @pl.when(i + 2 < n)
                def _():
                    start_in(i + 2, sl)

            def pair(t, _):
                for sl in range(2):
                    i = 2 * t + sl

                    @pl.when(i < n)
                    def _():
                        body(i, sl)

                return 0

            lax.fori_loop(0, (n + 1) // 2, pair, 0)

            for fsl in range(2):
                pend = ((n >= 1) & (((n - 1) % 2) == fsl)) | (
                    (n >= 2) & (((n - 2) % 2) == fsl))

                @pl.when(pend)
                def _(fsl=fsl):
                    wait_sc(fsl)

            plsc.subcore_barrier()

            pltpu.sync_copy(
                acc_sh.at[pl.ds(myrow0, TROWS)],
                out_hbm.at[pl.ds(r0 + myrow0, TROWS)])
            return 0

        lax.fori_loop(0, CPC, chunk, 0)

    return k


_sc_call = _mesh_kernel()


@jax.jit
def _run(meta, packed, vals, table):
    return _sc_call(meta, packed, vals, table)


def kernel(feature_embedding, field_idx, field_sub_idx, feature_idx,
           feature_vals, batch_idx):
    del field_sub_idx
    i32 = jnp.int32
    pad = NPAD - NNZ
    bi = jnp.concatenate([batch_idx.astype(i32), jnp.full((pad,), BATCH, i32)])
    fi = jnp.concatenate([field_idx.astype(i32), jnp.zeros((pad,), i32)])
    fx = jnp.concatenate([feature_idx.astype(i32), jnp.zeros((pad,), i32)])
    fv = jnp.concatenate(
        [feature_vals, jnp.zeros((pad,), jnp.float32)]).reshape(NB, K)
    packed = jnp.stack([bi, fi, fx]).reshape(3, NB, K).transpose(1, 0, 2)
    bounds = jnp.searchsorted(
        batch_idx,
        jnp.arange(0, BATCH + 1, CHUNK_B, dtype=i32)).astype(i32)
    c_of_w = jnp.arange(32, dtype=i32) % NCORE
    cols = CPC * c_of_w[:, None] + (jnp.arange(32, dtype=i32) // 8)[None, :]
    meta = bounds[jnp.minimum(cols, NCHUNK)]
    return _run(meta, packed, fv, feature_embedding)
